# baseline (speedup 1.0000x reference)
import functools

import jax
import jax.numpy as jnp
from jax import lax
from jax.experimental import pallas as pl
from jax.experimental.pallas import tpu as pltpu
from jax.experimental.pallas import tpu_sc as plsc

NUM_EDGE_TYPES = 44
DIM_EDGE = 16
E_TOTAL = 3_200_000

_info = plsc.get_sparse_core_info()
_NC, _NS = _info.num_cores, _info.num_subcores
_NW = _NC * _NS
_L = 16

_NBLK_TOTAL = E_TOTAL // 128
_BLK_LO = _NBLK_TOTAL // _NW
_NREM = _NBLK_TOTAL - _BLK_LO * _NW

_CBLK = 16
_CHUNK = _CBLK * 128
_TILE_W = 8 * 128
_HALF = _CBLK * _TILE_W
_NCHUNKS = 50
_NBUF = 2
_NOUTER = _NCHUNKS // _NBUF
_GROUPS = _CHUNK // _L

_REP_STRIDE = NUM_EDGE_TYPES * DIM_EDGE + 1
_REP_WORDS = _REP_STRIDE * _L


def _emb_kernel(idx_hbm, table_hbm, out_hbm, table_v, idx_v, rows_v,
                in_sem, out_sem, tab_s):
    wid = lax.axis_index("s") * _NC + lax.axis_index("c")
    nblk = _BLK_LO + jnp.where(wid < _NREM, 1, 0)
    wstart = _BLK_LO * wid + jnp.minimum(wid, _NREM)

    pltpu.async_copy(table_hbm, table_v, tab_s).wait()

    def chunk_start_blk(c):
        return wstart + jnp.minimum(c * _CBLK, nblk - _CBLK)

    def start_in(c, b):
        blk0 = chunk_start_blk(c)
        pltpu.async_copy(idx_hbm.at[pl.ds(blk0 * 128, _CHUNK)],
                         idx_v[b], in_sem[b])

    def wait_in(b):
        pltpu.make_async_copy(idx_hbm.at[pl.ds(0, _CHUNK)], idx_v[b],
                              in_sem[b]).wait()

    def start_out(c, b):
        blk0 = chunk_start_blk(c)
        for tr in range(2):
            pltpu.async_copy(
                rows_v[b].at[pl.ds(tr * _HALF, _HALF)],
                out_hbm.at[pl.ds((tr * _NBLK_TOTAL + blk0) * _TILE_W, _HALF)],
                out_sem[b])

    def wait_out(b):
        for tr in range(2):
            pltpu.make_async_copy(rows_v[b].at[pl.ds(tr * _HALF, _HALF)],
                                  out_hbm.at[pl.ds(0, _HALF)],
                                  out_sem[b]).wait()

    lane_base = lax.iota(jnp.int32, _L) * _REP_STRIDE
    dvecs = [lane_base + d for d in range(DIM_EDGE)]

    def compute(b):
        def blk_body(blk, _):
            ibase = pl.multiple_of(blk * 128, 128)
            obase = pl.multiple_of(blk * _TILE_W, _TILE_W)

            def iv_of(sg):
                return idx_v[b][pl.ds(ibase + sg * _L, _L)] * DIM_EDGE

            def store(sg, d, col):
                tr, dl = divmod(d, DIM_EDGE // 2)
                addr = pl.multiple_of(
                    obase + tr * _HALF + dl * 128 + sg * _L, _L)
                rows_v[b][pl.ds(addr, _L)] = col

            iv = iv_of(0)
            cols = [plsc.load_gather(table_v, [iv + dvecs[d]])
                    for d in range(DIM_EDGE)]
            for sg in range(1, 8):
                iv = iv_of(sg)
                new = []
                for d in range(DIM_EDGE):
                    new.append(plsc.load_gather(table_v, [iv + dvecs[d]]))
                    store(sg - 1, d, cols[d])
                cols = new
            for d in range(DIM_EDGE):
                store(7, d, cols[d])
            return ()
        lax.fori_loop(0, _CBLK, blk_body, (), unroll=2)

    for b in range(_NBUF):
        start_in(b, b)

    def outer_body(o, _):
        for b in range(_NBUF):
            c = o * _NBUF + b
            wait_in(b)

            @pl.when(o > 0)
            def _():
                wait_out(b)

            compute(b)
            start_out(c, b)

            @pl.when(o < _NOUTER - 1)
            def _():
                start_in(c + _NBUF, b)
        return ()

    lax.fori_loop(0, _NOUTER, outer_body, (), unroll=False)
    for b in range(_NBUF):
        wait_out(b)


def kernel(edge_attr, emb_weight):
    idx = edge_attr.astype(jnp.int32)
    table_flat = jnp.tile(
        jnp.concatenate([jnp.reshape(emb_weight, (-1,)),
                         jnp.zeros((1,), jnp.float32)]), _L)
    mesh = plsc.VectorSubcoreMesh(core_axis_name="c", subcore_axis_name="s")
    f = functools.partial(
        pl.kernel,
        out_type=jax.ShapeDtypeStruct((E_TOTAL * DIM_EDGE,), jnp.float32),
        mesh=mesh,
        scratch_types=[
            pltpu.VMEM((_REP_WORDS,), jnp.float32),
            [pltpu.VMEM((_CHUNK,), jnp.int32) for _ in range(_NBUF)],
            [pltpu.VMEM((2 * _HALF,), jnp.float32) for _ in range(_NBUF)],
            [pltpu.SemaphoreType.DMA for _ in range(_NBUF)],
            [pltpu.SemaphoreType.DMA for _ in range(_NBUF)],
            pltpu.SemaphoreType.DMA,
        ],
        compiler_params=pltpu.CompilerParams(
            use_tc_tiling_on_sc=False, needs_layout_passes=False
        ),
    )(_emb_kernel)
    flat = f(idx, table_flat)
    return (flat.reshape(2, _NBLK_TOTAL, 8, 128)
                .transpose(1, 3, 0, 2)
                .reshape(E_TOTAL, DIM_EDGE))


# speedup vs baseline: 1.0119x; 1.0119x over previous
import functools

import jax
import jax.numpy as jnp
from jax import lax
from jax.experimental import pallas as pl
from jax.experimental.pallas import tpu as pltpu
from jax.experimental.pallas import tpu_sc as plsc

NUM_EDGE_TYPES = 44
DIM_EDGE = 16
E_TOTAL = 3_200_000

_info = plsc.get_sparse_core_info()
_NC, _NS = _info.num_cores, _info.num_subcores
_NW = _NC * _NS
_L = 16

_NBLK_TOTAL = E_TOTAL // 128
_BLK_LO = _NBLK_TOTAL // _NW
_NREM = _NBLK_TOTAL - _BLK_LO * _NW

_CBLK = 16
_CHUNK = _CBLK * 128
_TILE_W = 8 * 128
_HALF = _CBLK * _TILE_W
_NCHUNKS = 50
_NBUF = 2
_NOUTER = _NCHUNKS // _NBUF
_GROUPS = _CHUNK // _L

_REP_STRIDE = NUM_EDGE_TYPES * DIM_EDGE + 1
_REP_WORDS = _REP_STRIDE * _L


def _emb_kernel(idx_hbm, table_hbm, out_hbm, table_v, idx_v, rows_v,
                in_sem, out_sem, tab_s):
    wid = lax.axis_index("s") * _NC + lax.axis_index("c")
    nblk = _BLK_LO + jnp.where(wid < _NREM, 1, 0)
    wstart = _BLK_LO * wid + jnp.minimum(wid, _NREM)

    pltpu.async_copy(table_hbm, table_v, tab_s).wait()

    def chunk_start_blk(c):
        return wstart + jnp.minimum(c * _CBLK, nblk - _CBLK)

    def start_in(c, b):
        blk0 = chunk_start_blk(c)
        pltpu.async_copy(idx_hbm.at[pl.ds(blk0 * 128, _CHUNK)],
                         idx_v[b], in_sem[b])

    def wait_in(b):
        pltpu.make_async_copy(idx_hbm.at[pl.ds(0, _CHUNK)], idx_v[b],
                              in_sem[b]).wait()

    def start_out(c, b):
        blk0 = chunk_start_blk(c)
        for tr in range(2):
            pltpu.async_copy(
                rows_v[b].at[pl.ds(tr * _HALF, _HALF)],
                out_hbm.at[pl.ds((tr * _NBLK_TOTAL + blk0) * _TILE_W, _HALF)],
                out_sem[b])

    def wait_out(b):
        for tr in range(2):
            pltpu.make_async_copy(rows_v[b].at[pl.ds(tr * _HALF, _HALF)],
                                  out_hbm.at[pl.ds(0, _HALF)],
                                  out_sem[b]).wait()

    lane_base = lax.iota(jnp.int32, _L) * _REP_STRIDE
    dvecs = [lane_base + d for d in range(DIM_EDGE)]

    def compute(b):
        def blk_body(blk, _):
            ibase = pl.multiple_of(blk * 128, 128)
            obase = pl.multiple_of(blk * _TILE_W, _TILE_W)

            def iv_of(sg):
                return idx_v[b][pl.ds(ibase + sg * _L, _L)] * DIM_EDGE

            def store(sg, d, col):
                tr, dl = divmod(d, DIM_EDGE // 2)
                addr = pl.multiple_of(
                    obase + tr * _HALF + dl * 128 + sg * _L, _L)
                rows_v[b][pl.ds(addr, _L)] = col

            iv = iv_of(0)
            cols = [plsc.load_gather(table_v, [iv + dvecs[d]])
                    for d in range(DIM_EDGE)]
            for sg in range(1, 8):
                iv = iv_of(sg)
                new = []
                for d in range(DIM_EDGE):
                    new.append(plsc.load_gather(table_v, [iv + dvecs[d]]))
                    store(sg - 1, d, cols[d])
                cols = new
            for d in range(DIM_EDGE):
                store(7, d, cols[d])
            return ()
        lax.fori_loop(0, _CBLK, blk_body, (), unroll=False)

    for b in range(_NBUF):
        start_in(b, b)

    def outer_body(o, _):
        for b in range(_NBUF):
            c = o * _NBUF + b
            wait_in(b)

            @pl.when(o > 0)
            def _():
                wait_out(b)

            compute(b)
            start_out(c, b)

            @pl.when(o < _NOUTER - 1)
            def _():
                start_in(c + _NBUF, b)
        return ()

    lax.fori_loop(0, _NOUTER, outer_body, (), unroll=False)
    for b in range(_NBUF):
        wait_out(b)


def kernel(edge_attr, emb_weight):
    idx = edge_attr.astype(jnp.int32)
    table_flat = jnp.tile(
        jnp.concatenate([jnp.reshape(emb_weight, (-1,)),
                         jnp.zeros((1,), jnp.float32)]), _L)
    mesh = plsc.VectorSubcoreMesh(core_axis_name="c", subcore_axis_name="s")
    f = functools.partial(
        pl.kernel,
        out_type=jax.ShapeDtypeStruct((E_TOTAL * DIM_EDGE,), jnp.float32),
        mesh=mesh,
        scratch_types=[
            pltpu.VMEM((_REP_WORDS,), jnp.float32),
            [pltpu.VMEM((_CHUNK,), jnp.int32) for _ in range(_NBUF)],
            [pltpu.VMEM((2 * _HALF,), jnp.float32) for _ in range(_NBUF)],
            [pltpu.SemaphoreType.DMA for _ in range(_NBUF)],
            [pltpu.SemaphoreType.DMA for _ in range(_NBUF)],
            pltpu.SemaphoreType.DMA,
        ],
        compiler_params=pltpu.CompilerParams(
            use_tc_tiling_on_sc=False, needs_layout_passes=False
        ),
    )(_emb_kernel)
    flat = f(idx, table_flat)
    return (flat.reshape(2, _NBLK_TOTAL, 8, 128)
                .transpose(1, 3, 0, 2)
                .reshape(E_TOTAL, DIM_EDGE))
